# initial kernel scaffold (unmeasured)
import jax
import jax.numpy as jnp
from jax import lax
from jax.experimental import pallas as pl
from jax.experimental.pallas import tpu as pltpu


def kernel(
    x,
):
    def body(*refs):
        pass

    out_shape = jax.ShapeDtypeStruct(..., jnp.float32)
    return pl.pallas_call(body, out_shape=out_shape)(...)


# baseline (device time: 191925 ns/iter reference)
import jax
import jax.numpy as jnp
from jax import lax
from jax.experimental import pallas as pl
from jax.experimental.pallas import tpu as pltpu


def kernel(x):
    _, m, n2 = x.shape
    nh = n2 // 2

    def body(x_hbm, out_ref, recv_buf, local_sem, send_sem, recv_sem):
        my_x = lax.axis_index("x")
        my_y = lax.axis_index("y")
        my_z = lax.axis_index("z")
        partner_y = 1 - my_y

        barrier_sem = pltpu.get_barrier_semaphore()
        pl.semaphore_signal(
            barrier_sem,
            inc=1,
            device_id=(my_x, partner_y, my_z),
            device_id_type=pl.DeviceIdType.MESH,
        )
        pl.semaphore_wait(barrier_sem, 1)

        local = pltpu.make_async_copy(
            x_hbm.at[0, :, pl.ds(my_y * nh, nh)], out_ref, local_sem
        )
        local.start()

        rdma = pltpu.make_async_remote_copy(
            src_ref=x_hbm.at[0, :, pl.ds(partner_y * nh, nh)],
            dst_ref=recv_buf,
            send_sem=send_sem,
            recv_sem=recv_sem,
            device_id=(my_x, partner_y, my_z),
            device_id_type=pl.DeviceIdType.MESH,
        )
        rdma.start()

        local.wait()
        rdma.wait()
        out_ref[...] += recv_buf[...]

    return pl.pallas_call(
        body,
        out_shape=jax.ShapeDtypeStruct((m, nh), jnp.float32),
        in_specs=[pl.BlockSpec(memory_space=pl.ANY)],
        out_specs=pl.BlockSpec(memory_space=pltpu.VMEM),
        scratch_shapes=[
            pltpu.VMEM((m, nh), jnp.float32),
            pltpu.SemaphoreType.DMA,
            pltpu.SemaphoreType.DMA,
            pltpu.SemaphoreType.DMA,
        ],
        compiler_params=pltpu.CompilerParams(collective_id=0),
    )(x)


# device time: 94610 ns/iter; 2.0286x vs baseline; 2.0286x over previous
import jax
import jax.numpy as jnp
from jax import lax
from jax.experimental import pallas as pl
from jax.experimental.pallas import tpu as pltpu

C = 4


def kernel(x):
    _, m, n2 = x.shape
    nh = n2 // 2
    q = m // 4
    cr = q // C
    hr = cr // 2

    def body(
        x_hbm, out_ref, rbuf, local_sem,
        sy, sx, sz, sxh, szh,
        ry, rx, rz, rxd, rzd,
    ):
        my_x = lax.axis_index("x")
        my_y = lax.axis_index("y")
        my_z = lax.axis_index("z")
        ypartner = (my_x, 1 - my_y, my_z)
        xnbr = (1 - my_x, my_y, my_z)
        znbr = (my_x, my_y, 1 - my_z)

        my_p = 2 * my_x + my_z
        px = 2 * (1 - my_x) + my_z
        pz = 2 * my_x + (1 - my_z)

        def rows(p, c, off=0, nrows=cr):
            return pl.ds(p * q + c * cr + off, nrows)

        barrier_sem = pltpu.get_barrier_semaphore()
        for nbr in (ypartner, xnbr, znbr):
            pl.semaphore_signal(
                barrier_sem, inc=1, device_id=nbr,
                device_id_type=pl.DeviceIdType.MESH,
            )
        pl.semaphore_wait(barrier_sem, 3)

        local = pltpu.make_async_copy(
            x_hbm.at[0, :, pl.ds(my_y * nh, nh)], out_ref, local_sem
        )
        local.start()

        def rdma(src, dst, ssem, rsem, dev):
            return pltpu.make_async_remote_copy(
                src_ref=src, dst_ref=dst, send_sem=ssem, recv_sem=rsem,
                device_id=dev, device_id_type=pl.DeviceIdType.MESH,
            )

        for c in range(C):
            rdma(
                x_hbm.at[0, rows(my_p, c), pl.ds((1 - my_y) * nh, nh)],
                rbuf.at[rows(my_p, c), :],
                sy.at[c], ry.at[c], ypartner,
            ).start()

        for c in range(C):
            rdma(
                rbuf.at[rows(my_p, c), :], rbuf.at[rows(my_p, c), :],
                sy.at[c], ry.at[c], ypartner,
            ).wait_recv()
            rdma(
                rbuf.at[rows(my_p, c), :], rbuf.at[rows(my_p, c), :],
                sx.at[c], rx.at[c], xnbr,
            ).start()
            rdma(
                rbuf.at[rows(my_p, c), :], rbuf.at[rows(my_p, c), :],
                sz.at[c], rz.at[c], znbr,
            ).start()

        for c in range(C):
            rdma(
                rbuf.at[rows(px, c), :], rbuf.at[rows(px, c), :],
                sx.at[c], rx.at[c], xnbr,
            ).wait_recv()
            rdma(
                rbuf.at[rows(px, c, hr, hr), :],
                rbuf.at[rows(px, c, hr, hr), :],
                szh.at[c], rzd.at[c], znbr,
            ).start()
            rdma(
                rbuf.at[rows(pz, c), :], rbuf.at[rows(pz, c), :],
                sz.at[c], rz.at[c], znbr,
            ).wait_recv()
            rdma(
                rbuf.at[rows(pz, c, 0, hr), :],
                rbuf.at[rows(pz, c, 0, hr), :],
                sxh.at[c], rxd.at[c], xnbr,
            ).start()

        pd = 2 * (1 - my_x) + (1 - my_z)
        for c in range(C):
            rdma(
                rbuf.at[rows(pd, c, 0, hr), :], rbuf.at[rows(pd, c, 0, hr), :],
                sxh.at[c], rxd.at[c], xnbr,
            ).wait_recv()
            rdma(
                rbuf.at[rows(pd, c, hr, hr), :], rbuf.at[rows(pd, c, hr, hr), :],
                szh.at[c], rzd.at[c], znbr,
            ).wait_recv()

        local.wait()
        out_ref[...] += rbuf[...]

        for c in range(C):
            rdma(
                x_hbm.at[0, rows(my_p, c), pl.ds((1 - my_y) * nh, nh)],
                rbuf.at[rows(my_p, c), :], sy.at[c], ry.at[c], ypartner,
            ).wait_send()
            rdma(
                rbuf.at[rows(my_p, c), :], rbuf.at[rows(my_p, c), :],
                sx.at[c], rx.at[c], xnbr,
            ).wait_send()
            rdma(
                rbuf.at[rows(my_p, c), :], rbuf.at[rows(my_p, c), :],
                sz.at[c], rz.at[c], znbr,
            ).wait_send()
            rdma(
                rbuf.at[rows(px, c, hr, hr), :], rbuf.at[rows(px, c, hr, hr), :],
                szh.at[c], rzd.at[c], znbr,
            ).wait_send()
            rdma(
                rbuf.at[rows(pz, c, 0, hr), :], rbuf.at[rows(pz, c, 0, hr), :],
                sxh.at[c], rxd.at[c], xnbr,
            ).wait_send()

    return pl.pallas_call(
        body,
        out_shape=jax.ShapeDtypeStruct((m, nh), jnp.float32),
        in_specs=[pl.BlockSpec(memory_space=pl.ANY)],
        out_specs=pl.BlockSpec(memory_space=pltpu.VMEM),
        scratch_shapes=[
            pltpu.VMEM((m, nh), jnp.float32),
            pltpu.SemaphoreType.DMA,
            pltpu.SemaphoreType.DMA((C,)),
            pltpu.SemaphoreType.DMA((C,)),
            pltpu.SemaphoreType.DMA((C,)),
            pltpu.SemaphoreType.DMA((C,)),
            pltpu.SemaphoreType.DMA((C,)),
            pltpu.SemaphoreType.DMA((C,)),
            pltpu.SemaphoreType.DMA((C,)),
            pltpu.SemaphoreType.DMA((C,)),
            pltpu.SemaphoreType.DMA((C,)),
            pltpu.SemaphoreType.DMA((C,)),
        ],
        compiler_params=pltpu.CompilerParams(collective_id=0),
    )(x)
